# baseline (speedup 1.0000x reference)
import functools

import jax
import jax.numpy as jnp
from jax.experimental import pallas as pl
from jax.experimental.pallas import tpu as pltpu

_S = 128
_K = 100
_KPAD = 128
_L = 8192
_NUM_NEGS = 20


def _knn_kernel(q_ref, loc_ref, choice_ref, out_ref, bd_ref, bi_ref,
                *, nsteps):
    i = pl.program_id(0)

    q = q_ref[...]
    loc = loc_ref[...]
    qx = q[:, 0:1]
    qy = q[:, 1:2]
    lx = loc[0:1, :]
    ly = loc[1:2, :]

    qq = qx * qx + qy * qy
    ll = lx * lx + ly * ly
    dot = jnp.dot(q, loc, preferred_element_type=jnp.float32)
    d2 = (qq - 2.0 * dot) + ll
    negd = -d2

    base = i * _L
    lane = jax.lax.broadcasted_iota(jnp.int32, (_S, _L), 1)
    col = jax.lax.broadcasted_iota(jnp.int32, (_S, _KPAD), 1)

    def _extract(it, carry):
        nd, td, ti = carry
        m = jnp.max(nd, axis=1, keepdims=True)
        p = jnp.min(jnp.where(nd == m, lane, _L), axis=1, keepdims=True)
        td = jnp.where(col == it, m, td)
        ti = jnp.where(col == it, p + base, ti)
        nd = jnp.where(lane == p, -jnp.inf, nd)
        return nd, td, ti

    td0 = jnp.zeros((_S, _KPAD), jnp.float32)
    ti0 = jnp.zeros((_S, _KPAD), jnp.int32)
    _, td, ti = jax.lax.fori_loop(0, _KPAD, _extract, (negd, td0, ti0))
    bd_ref[:, pl.ds(i * _KPAD, _KPAD)] = td
    bi_ref[:, pl.ds(i * _KPAD, _KPAD)] = ti

    @pl.when(i == nsteps - 1)
    def _finalize():
        ncand = nsteps * _KPAD
        choice = choice_ref[...]
        iota_c = jax.lax.broadcasted_iota(jnp.int32, (_S, ncand), 1)
        bi = bi_ref[...]

        def _merge(rank, carry):
            bd, acc = carry
            m2 = jnp.max(bd, axis=1, keepdims=True)
            p2 = jnp.min(jnp.where(bd == m2, iota_c, ncand), axis=1, keepdims=True)
            gi = jnp.sum(jnp.where(iota_c == p2, bi, 0), axis=1, keepdims=True)
            acc = jnp.where(choice == rank, gi, acc)
            bd = jnp.where(iota_c == p2, -jnp.inf, bd)
            return bd, acc

        acc0 = jnp.zeros((_S, _NUM_NEGS), jnp.int32)
        _, acc = jax.lax.fori_loop(0, _K, _merge, (bd_ref[...], acc0))
        out_ref[...] = acc


@jax.jit
def _run(q, loc_t, choice_idx):
    npad = loc_t.shape[1]
    nsteps = npad // _L
    grid = (nsteps,)
    kern = functools.partial(_knn_kernel, nsteps=nsteps)
    return pl.pallas_call(
        kern,
        grid=grid,
        in_specs=[
            pl.BlockSpec((_S, 2), lambda i: (0, 0)),
            pl.BlockSpec((2, _L), lambda i: (0, i)),
            pl.BlockSpec((_S, _NUM_NEGS), lambda i: (0, 0)),
        ],
        out_specs=pl.BlockSpec((_S, _NUM_NEGS), lambda i: (0, 0)),
        out_shape=jax.ShapeDtypeStruct((_S, _NUM_NEGS), jnp.int32),
        scratch_shapes=[
            pltpu.VMEM((_S, nsteps * _KPAD), jnp.float32),
            pltpu.VMEM((_S, nsteps * _KPAD), jnp.int32),
        ],
    )(q, loc_t, choice_idx)


def kernel(trg_seq, loc_coords, num_negs):
    n_locs = loc_coords.shape[0]
    trg_locs = trg_seq[:, 1].astype(jnp.int32)
    q = loc_coords[trg_locs]

    nsteps = -(-n_locs // _L)
    npad = nsteps * _L
    loc_pad = jnp.full((npad - n_locs, 2), 100.0, dtype=loc_coords.dtype)
    loc_t = jnp.concatenate([loc_coords, loc_pad], axis=0).T

    skey = jax.random.key(42)
    choice_idx = jax.random.randint(skey, (trg_locs.shape[0], _NUM_NEGS), 0, _K)
    choice_idx = choice_idx.astype(jnp.int32)

    return _run(q, loc_t, choice_idx)


# speedup vs baseline: 1.2681x; 1.2681x over previous
import functools

import jax
import jax.numpy as jnp
from jax.experimental import pallas as pl
from jax.experimental.pallas import tpu as pltpu

_S = 128
_K = 100
_KPAD = 128
_L = 8192
_NUM_NEGS = 20


def _knn_kernel(q_ref, loc_ref, choice_ref, out_ref, bd_ref, bi_ref,
                *, nsteps):
    i = pl.program_id(0)

    q = q_ref[...]
    loc = loc_ref[...]
    qx = q[:, 0:1]
    qy = q[:, 1:2]
    lx = loc[0:1, :]
    ly = loc[1:2, :]

    qq = qx * qx + qy * qy
    ll = lx * lx + ly * ly
    dot = jnp.dot(q, loc, preferred_element_type=jnp.float32)
    d2 = (qq - 2.0 * dot) + ll
    negd = -d2

    base = i * _L
    lane = jax.lax.broadcasted_iota(jnp.int32, (_S, _L), 1)
    col = jax.lax.broadcasted_iota(jnp.int32, (_S, _KPAD), 1)

    def _extract(it, carry):
        nd, td, ti = carry
        m = jnp.max(nd, axis=1, keepdims=True)
        p = jnp.min(jnp.where(nd == m, lane, _L), axis=1, keepdims=True)
        td = jnp.where(col == it, m, td)
        ti = jnp.where(col == it, p + base, ti)
        nd = jnp.where(lane == p, -jnp.inf, nd)
        return nd, td, ti

    td0 = jnp.full((_S, _KPAD), -jnp.inf, jnp.float32)
    ti0 = jnp.zeros((_S, _KPAD), jnp.int32)
    _, td, ti = jax.lax.fori_loop(0, _K, _extract, (negd, td0, ti0))
    bd_ref[:, pl.ds(i * _KPAD, _KPAD)] = td
    bi_ref[:, pl.ds(i * _KPAD, _KPAD)] = ti

    @pl.when(i == nsteps - 1)
    def _finalize():
        ncand = nsteps * _KPAD
        choice = choice_ref[...]
        iota_c = jax.lax.broadcasted_iota(jnp.int32, (_S, ncand), 1)
        bi = bi_ref[...]

        def _merge(rank, carry):
            bd, acc = carry
            m2 = jnp.max(bd, axis=1, keepdims=True)
            p2 = jnp.min(jnp.where(bd == m2, iota_c, ncand), axis=1, keepdims=True)
            gi = jnp.sum(jnp.where(iota_c == p2, bi, 0), axis=1, keepdims=True)
            acc = jnp.where(choice == rank, gi, acc)
            bd = jnp.where(iota_c == p2, -jnp.inf, bd)
            return bd, acc

        acc0 = jnp.zeros((_S, _NUM_NEGS), jnp.int32)
        _, acc = jax.lax.fori_loop(0, _K, _merge, (bd_ref[...], acc0))
        out_ref[...] = acc


@jax.jit
def _run(q, loc_t, choice_idx):
    npad = loc_t.shape[1]
    nsteps = npad // _L
    grid = (nsteps,)
    kern = functools.partial(_knn_kernel, nsteps=nsteps)
    return pl.pallas_call(
        kern,
        grid=grid,
        in_specs=[
            pl.BlockSpec((_S, 2), lambda i: (0, 0)),
            pl.BlockSpec((2, _L), lambda i: (0, i)),
            pl.BlockSpec((_S, _NUM_NEGS), lambda i: (0, 0)),
        ],
        out_specs=pl.BlockSpec((_S, _NUM_NEGS), lambda i: (0, 0)),
        out_shape=jax.ShapeDtypeStruct((_S, _NUM_NEGS), jnp.int32),
        scratch_shapes=[
            pltpu.VMEM((_S, nsteps * _KPAD), jnp.float32),
            pltpu.VMEM((_S, nsteps * _KPAD), jnp.int32),
        ],
    )(q, loc_t, choice_idx)


def kernel(trg_seq, loc_coords, num_negs):
    n_locs = loc_coords.shape[0]
    trg_locs = trg_seq[:, 1].astype(jnp.int32)
    q = loc_coords[trg_locs]

    nsteps = -(-n_locs // _L)
    npad = nsteps * _L
    loc_pad = jnp.full((npad - n_locs, 2), 100.0, dtype=loc_coords.dtype)
    loc_t = jnp.concatenate([loc_coords, loc_pad], axis=0).T

    skey = jax.random.key(42)
    choice_idx = jax.random.randint(skey, (trg_locs.shape[0], _NUM_NEGS), 0, _K)
    choice_idx = choice_idx.astype(jnp.int32)

    return _run(q, loc_t, choice_idx)
